# baseline (speedup 1.0000x reference)
import functools

import jax
import jax.numpy as jnp
from jax import lax
from jax.experimental import pallas as pl
from jax.experimental.pallas import tpu as pltpu
from jax.experimental.pallas import tpu_sc as plsc

_N_CAT = 26
_NUM_CATS = 1_000_000
_BATCH = 16384
_CAT0 = 13
_NC, _NS = 2, 16
_NW = _NC * _NS
_RPW = _BATCH // _NW
_CHUNK = 128
_NCH = _RPW // _CHUNK
_UNROLL = 8


def _sc_gather(emb, idx4):
    mesh = plsc.VectorSubcoreMesh(core_axis_name="c", subcore_axis_name="s")

    @functools.partial(
        pl.kernel,
        out_type=jax.ShapeDtypeStruct((_NW, _N_CAT, _NCH, _CHUNK), jnp.float32),
        mesh=mesh,
        scratch_types=[
            pltpu.VMEM((_N_CAT, _NCH, _CHUNK), jnp.float32),
            pltpu.VMEM_SHARED((_NS, _N_CAT, _NCH, _CHUNK), jnp.int32),
            pltpu.SMEM((_CHUNK,), jnp.int32),
            pltpu.SemaphoreType.DMA,
        ],
    )
    def k(emb_hbm, idx_hbm, out_hbm, dst_v, idx_sh, idx_s, sem):
        sid = lax.axis_index("s")
        wid = sid * _NC + lax.axis_index("c")
        pltpu.sync_copy(idx_hbm.at[wid], idx_sh.at[sid])

        def col_body(i, carry):
            def chunk_body(c, carry2):
                pltpu.sync_copy(idx_sh.at[sid, i, c], idx_s)

                def elem_body(t, carry3):
                    base = t * _UNROLL
                    for u in range(_UNROLL):
                        r = idx_s[base + u]
                        pltpu.async_copy(
                            emb_hbm.at[i, pl.ds(r, 1), 0],
                            dst_v.at[i, c, pl.ds(base + u, 1)],
                            sem,
                        )
                    return carry3

                lax.fori_loop(0, _CHUNK // _UNROLL, elem_body, 0)
                pltpu.make_async_copy(
                    out_hbm.at[wid, i, c], dst_v.at[i, c], sem
                ).wait()
                return carry2

            lax.fori_loop(0, _NCH, chunk_body, 0)
            return carry

        lax.fori_loop(0, _N_CAT, col_body, 0)
        pltpu.sync_copy(dst_v, out_hbm.at[wid])

    return k(emb, idx4)


def kernel(inputs, embeddings):
    cats = inputs[:, _CAT0:].astype(jnp.int32)
    idx4 = cats.reshape(_NW, _NCH, _CHUNK, _N_CAT).transpose(0, 3, 1, 2)
    g = _sc_gather(embeddings, idx4)
    gathered = g.reshape(_NW, _N_CAT, _RPW).transpose(0, 2, 1).reshape(_BATCH, _N_CAT)
    return jnp.concatenate([inputs[:, :_CAT0], gathered], axis=1)


# speedup vs baseline: 2.3958x; 2.3958x over previous
import functools

import jax
import jax.numpy as jnp
from jax import lax
from jax.experimental import pallas as pl
from jax.experimental.pallas import tpu as pltpu
from jax.experimental.pallas import tpu_sc as plsc

_N_CAT = 26
_NUM_CATS = 1_000_000
_BATCH = 16384
_CAT0 = 13
_TOT = _BATCH * _N_CAT
_NC, _NS = 2, 16
_NW = _NC * _NS
_PER_W = _TOT // _NW
_CHUNK = 128
_NCH = _PER_W // _CHUNK
_FIRE = 8
_NLOOP = _NCH // _FIRE


def _sc_gather(table, idx3):
    mesh = plsc.VectorSubcoreMesh(core_axis_name="c", subcore_axis_name="s")

    @functools.partial(
        pl.kernel,
        out_type=jax.ShapeDtypeStruct((_NW, _NCH, _CHUNK), jnp.float32),
        mesh=mesh,
        scratch_types=[
            pltpu.VMEM((_NCH, _CHUNK), jnp.int32),
            pltpu.VMEM((_NCH, _CHUNK), jnp.float32),
            pltpu.SemaphoreType.DMA,
        ],
    )
    def k(table_hbm, idx_hbm, out_hbm, idx_v, dst_v, sem):
        wid = lax.axis_index("s") * _NC + lax.axis_index("c")
        pltpu.sync_copy(idx_hbm.at[wid], idx_v)

        def body(o, carry):
            base = o * _FIRE
            descs = [
                pltpu.async_copy(
                    table_hbm.at[idx_v.at[base + j]], dst_v.at[base + j], sem
                )
                for j in range(_FIRE)
            ]
            for d in descs:
                d.wait()
            return carry

        lax.fori_loop(0, _NLOOP, body, 0)
        pltpu.sync_copy(dst_v, out_hbm.at[wid])

    return k(table, idx3)


def kernel(inputs, embeddings):
    table = embeddings[:, :, 0].reshape(-1)
    offs = jnp.arange(_N_CAT, dtype=jnp.int32) * _NUM_CATS
    idx = inputs[:, _CAT0:].astype(jnp.int32) + offs[None, :]
    gathered = _sc_gather(table, idx.reshape(_NW, _NCH, _CHUNK))
    return jnp.concatenate(
        [inputs[:, :_CAT0], gathered.reshape(_BATCH, _N_CAT)], axis=1
    )


# speedup vs baseline: 2.3967x; 1.0004x over previous
import functools

import jax
import jax.numpy as jnp
from jax import lax
from jax.experimental import pallas as pl
from jax.experimental.pallas import tpu as pltpu
from jax.experimental.pallas import tpu_sc as plsc

_N_CAT = 26
_NUM_CATS = 1_000_000
_BATCH = 16384
_CAT0 = 13
_TOT = _BATCH * _N_CAT
_NC, _NS = 2, 16
_NW = _NC * _NS
_PER_W = _TOT // _NW
_CHUNK = 128
_NCH = _PER_W // _CHUNK
_FIRE = 8
_NLOOP = _NCH // _FIRE


def _sc_gather(table, idx3):
    mesh = plsc.VectorSubcoreMesh(core_axis_name="c", subcore_axis_name="s")

    @functools.partial(
        pl.kernel,
        out_type=jax.ShapeDtypeStruct((_NW, _NCH, _CHUNK), jnp.float32),
        mesh=mesh,
        scratch_types=[
            pltpu.VMEM((_NCH, _CHUNK), jnp.int32),
            pltpu.VMEM((_NCH, _CHUNK), jnp.float32),
            pltpu.SemaphoreType.DMA,
        ],
    )
    def k(table_hbm, idx_hbm, out_hbm, idx_v, dst_v, sem):
        wid = lax.axis_index("s") * _NC + lax.axis_index("c")
        pltpu.sync_copy(idx_hbm.at[wid], idx_v)

        def body(o, carry):
            base = o * _FIRE
            descs = [
                pltpu.async_copy(
                    table_hbm.at[idx_v.at[base + j]], dst_v.at[base + j], sem
                )
                for j in range(_FIRE)
            ]
            for d in descs:
                d.wait()
            return carry

        lax.fori_loop(0, _NLOOP, body, 0)
        pltpu.sync_copy(dst_v, out_hbm.at[wid])

    return k(table, idx3)


def kernel(inputs, embeddings):
    table = embeddings.reshape(-1)
    offs = jnp.arange(_N_CAT, dtype=jnp.int32) * _NUM_CATS
    idx = inputs[:, _CAT0:].astype(jnp.int32) + offs[None, :]
    gathered = _sc_gather(table, idx.reshape(_NW, _NCH, _CHUNK))
    return jnp.concatenate(
        [inputs[:, :_CAT0], gathered.reshape(_BATCH, _N_CAT)], axis=1
    )
